# Initial kernel scaffold; baseline (speedup 1.0000x reference)
import jax
import jax.numpy as jnp
from jax.experimental import pallas as pl


def kernel(x, batch, W_gamma, b_gamma, W_lambda):
    raise NotImplementedError("write your pallas kernel here")


# speedup vs baseline: 1.0683x; 1.0683x over previous
import jax
import jax.numpy as jnp
from jax.experimental import pallas as pl
from jax.experimental.pallas import tpu as pltpu

NUM_SEGMENTS = 10000
BLK = 256


def _body(batch_ref, x_ref, wg_ref, bg_ref, wl_ref, out_ref, acc_ref, carry_ref):
    phase = pl.program_id(0)
    b = pl.program_id(1)
    B = BLK

    bvec = batch_ref[0]
    b0 = batch_ref[0, 0, 0]
    blast = batch_ref[0, 0, B - 1]

    prev_last = jnp.where(b == 0, b0, carry_ref[1])
    carry_rank = jnp.where(b == 0, 0, carry_ref[0])

    lane = jax.lax.broadcasted_iota(jnp.int32, (1, B), 1)
    rolled = pltpu.roll(bvec, 1, 1)
    shifted = jnp.where(lane == 0, prev_last, rolled)
    newseg = (bvec != shifted).astype(jnp.float32)

    tri = (jax.lax.broadcasted_iota(jnp.int32, (B, B), 0)
           <= jax.lax.broadcasted_iota(jnp.int32, (B, B), 1)).astype(jnp.float32)
    cum = jax.lax.dot_general(newseg, tri, (((1,), (0,)), ((), ())),
                              preferred_element_type=jnp.float32)

    ns0 = (b0 != prev_last).astype(jnp.float32)
    rel = cum - ns0
    rstart = carry_rank + (b0 != prev_last).astype(jnp.int32)
    nseg = jnp.sum(newseg).astype(jnp.int32)

    reli = rel.astype(jnp.int32)
    ohT = (jax.lax.broadcasted_iota(jnp.int32, (B, B), 0)
           == jnp.broadcast_to(reli, (B, B))).astype(jnp.float32)

    @pl.when(jnp.logical_and(phase == 0, b == 0))
    def _init():
        acc_ref[...] = jnp.zeros_like(acc_ref)

    @pl.when(phase == 0)
    def _accumulate():
        xb = x_ref[...]
        partial = jax.lax.dot_general(ohT, xb, (((1,), (0,)), ((), ())),
                                      preferred_element_type=jnp.float32)
        acc_ref[pl.ds(rstart, B), :] += partial

    @pl.when(jnp.logical_and(phase == 1, b == 0))
    def _transform():
        wl = wl_ref[...]

        def chunk(c, _):
            blk = acc_ref[pl.ds(c * B, B), :]
            acc_ref[pl.ds(c * B, B), :] = jax.lax.dot_general(
                blk, wl, (((1,), (1,)), ((), ())),
                preferred_element_type=jnp.float32)
            return 0

        jax.lax.fori_loop(0, acc_ref.shape[0] // B, chunk, 0)

    @pl.when(phase == 1)
    def _emit():
        xb = x_ref[...]
        ymw = acc_ref[pl.ds(rstart, B), :]
        expand = jax.lax.dot_general(ohT, ymw, (((0,), (0,)), ((), ())),
                                     preferred_element_type=jnp.float32)
        xg = jax.lax.dot_general(xb, wg_ref[...], (((1,), (1,)), ((), ())),
                                 preferred_element_type=jnp.float32)
        out_ref[...] = xg + bg_ref[...] - expand

    carry_ref[0] = carry_rank + nseg
    carry_ref[1] = blast


def kernel(x, batch, W_gamma, b_gamma, W_lambda):
    N, D = x.shape
    B = BLK
    NB = N // B
    acc_rows = ((NUM_SEGMENTS + B + B - 1) // B) * B

    batch3 = batch.astype(jnp.int32).reshape(NB, 1, B)
    bg2 = b_gamma.reshape(1, D)

    grid = (2, NB)
    out = pl.pallas_call(
        _body,
        grid=grid,
        in_specs=[
            pl.BlockSpec((1, 1, B), lambda p, b: (b, 0, 0)),
            pl.BlockSpec((B, D), lambda p, b: (b, 0)),
            pl.BlockSpec((D, D), lambda p, b: (0, 0)),
            pl.BlockSpec((1, D), lambda p, b: (0, 0)),
            pl.BlockSpec((D, D), lambda p, b: (0, 0)),
        ],
        out_specs=pl.BlockSpec((B, D), lambda p, b: (jnp.where(p == 0, 0, b), 0)),
        out_shape=jax.ShapeDtypeStruct((N, D), jnp.float32),
        scratch_shapes=[
            pltpu.VMEM((acc_rows, D), jnp.float32),
            pltpu.SMEM((2,), jnp.int32),
        ],
        compiler_params=pltpu.CompilerParams(
            dimension_semantics=("arbitrary", "arbitrary"),
        ),
    )(batch3, x, W_gamma, bg2, W_lambda)
    return out


# speedup vs baseline: 2.5560x; 2.3926x over previous
import jax
import jax.numpy as jnp
from jax import lax
from jax.experimental import pallas as pl
from jax.experimental.pallas import tpu as pltpu
from jax.experimental.pallas import tpu_sc as plsc

S = 10000
SP = 10240
D = 128
NC, NS = 2, 16
NW = NC * NS
R = 200
IW = 100
SEG_PER_SUB = SP // NS
ZR = 128


def _zero_rows(buf, rows):
    z = jnp.zeros((16,), jnp.float32)

    def body(i, _):
        r = i // (D // 16)
        k = i % (D // 16)
        buf[r, pl.ds(k * 16, 16)] = z
        return 0

    lax.fori_loop(0, rows * (D // 16), body, 0)


def _segsum_body(x_hbm, batch_hbm, out_hbm, xbuf, idxall, acc_sh):
    c = lax.axis_index("c")
    s = lax.axis_index("s")
    n = x_hbm.shape[0]
    worker_rows = n // NW
    chunks = worker_rows // R
    w = c * NS + s
    base0 = w * worker_rows

    _zero_rows(xbuf, ZR)
    for z in range(SEG_PER_SUB // ZR):
        pltpu.sync_copy(xbuf.at[pl.ds(0, ZR)],
                        acc_sh.at[pl.ds(s * SEG_PER_SUB + z * ZR, ZR)])
    pltpu.sync_copy(batch_hbm.at[w], idxall)
    plsc.subcore_barrier()

    def chunk(t, _):
        base = base0 + t * R
        pltpu.sync_copy(x_hbm.at[pl.ds(base, R)], xbuf)
        for j in range(R // IW):
            pltpu.sync_copy(xbuf.at[pl.ds(j * IW, IW)],
                            acc_sh.at[idxall.at[t * (R // IW) + j]], add=True)
        return 0

    lax.fori_loop(0, chunks, chunk, 0)
    plsc.subcore_barrier()
    pltpu.sync_copy(acc_sh.at[pl.ds(s * SEG_PER_SUB, SEG_PER_SUB)],
                    out_hbm.at[c, pl.ds(s * SEG_PER_SUB, SEG_PER_SUB)])


def _gather_body(ym_hbm, batch_hbm, out_hbm, gbuf, idxall, ym_sh):
    c = lax.axis_index("c")
    s = lax.axis_index("s")
    n = out_hbm.shape[0]
    worker_rows = n // NW
    chunks = worker_rows // R
    w = c * NS + s
    base0 = w * worker_rows

    pltpu.sync_copy(ym_hbm.at[pl.ds(s * SEG_PER_SUB, SEG_PER_SUB)],
                    ym_sh.at[pl.ds(s * SEG_PER_SUB, SEG_PER_SUB)])
    pltpu.sync_copy(batch_hbm.at[w], idxall)
    plsc.subcore_barrier()

    def chunk(t, _):
        base = base0 + t * R
        for j in range(R // IW):
            pltpu.sync_copy(ym_sh.at[idxall.at[t * (R // IW) + j]],
                            gbuf.at[pl.ds(j * IW, IW)])
        pltpu.sync_copy(gbuf, out_hbm.at[pl.ds(base, R)])
        return 0

    lax.fori_loop(0, chunks, chunk, 0)


def _lambda_body(xm_ref, wl_ref, ym_ref):
    xm = xm_ref[0] + xm_ref[1]
    ym_ref[...] = lax.dot_general(xm, wl_ref[...], (((1,), (1,)), ((), ())),
                                  preferred_element_type=jnp.float32)


def _gamma_body(x_ref, ye_ref, wg_ref, bg_ref, out_ref):
    xg = lax.dot_general(x_ref[...], wg_ref[...], (((1,), (1,)), ((), ())),
                         preferred_element_type=jnp.float32)
    out_ref[...] = xg + bg_ref[...] - ye_ref[...]


def kernel(x, batch, W_gamma, b_gamma, W_lambda):
    N = x.shape[0]
    per_w = N // NW
    batch3d = batch.astype(jnp.int32).reshape(NW, per_w // IW, IW)
    mesh = plsc.VectorSubcoreMesh(core_axis_name="c", subcore_axis_name="s",
                                  num_cores=NC, num_subcores=NS)

    seg = pl.kernel(
        _segsum_body,
        out_type=jax.ShapeDtypeStruct((NC, SP, D), jnp.float32),
        mesh=mesh,
        scratch_types=[
            pltpu.VMEM((R, D), jnp.float32),
            pltpu.VMEM((per_w // IW, IW), jnp.int32),
            pltpu.VMEM_SHARED((SP, D), jnp.float32),
        ],
    )
    xm2 = seg(x, batch3d)

    B2 = 1024
    ym = pl.pallas_call(
        _lambda_body,
        grid=(SP // B2,),
        in_specs=[
            pl.BlockSpec((2, B2, D), lambda i: (0, i, 0)),
            pl.BlockSpec((D, D), lambda i: (0, 0)),
        ],
        out_specs=pl.BlockSpec((B2, D), lambda i: (i, 0)),
        out_shape=jax.ShapeDtypeStruct((SP, D), jnp.float32),
    )(xm2, W_lambda)

    gat = pl.kernel(
        _gather_body,
        out_type=jax.ShapeDtypeStruct((N, D), jnp.float32),
        mesh=mesh,
        scratch_types=[
            pltpu.VMEM((R, D), jnp.float32),
            pltpu.VMEM((per_w // IW, IW), jnp.int32),
            pltpu.VMEM_SHARED((SP, D), jnp.float32),
        ],
    )
    yexp = gat(ym, batch3d)

    B4 = 512
    out = pl.pallas_call(
        _gamma_body,
        grid=(N // B4,),
        in_specs=[
            pl.BlockSpec((B4, D), lambda i: (i, 0)),
            pl.BlockSpec((B4, D), lambda i: (i, 0)),
            pl.BlockSpec((D, D), lambda i: (0, 0)),
            pl.BlockSpec((1, D), lambda i: (0, 0)),
        ],
        out_specs=pl.BlockSpec((B4, D), lambda i: (i, 0)),
        out_shape=jax.ShapeDtypeStruct((N, D), jnp.float32),
    )(x, yexp, W_gamma, b_gamma.reshape(1, D))
    return out
